# initial kernel scaffold (unmeasured)
import jax
import jax.numpy as jnp
from jax import lax
from jax.experimental import pallas as pl
from jax.experimental.pallas import tpu as pltpu


def kernel(
    x,
):
    def body(*refs):
        pass

    out_shape = jax.ShapeDtypeStruct(..., jnp.float32)
    return pl.pallas_call(body, out_shape=out_shape)(...)


# baseline (device time: 85304 ns/iter reference)
import jax
import jax.numpy as jnp
from jax import lax
from jax.experimental import pallas as pl
from jax.experimental.pallas import tpu as pltpu

N_DEV = 4
N_HOPS = 2 * (N_DEV - 1)


def kernel(x):
    m, n = x.shape
    m_c = m // N_DEV
    n_h = n // 2

    def body(
        x_ref,
        out_ref,
        comm_cw,
        comm_ccw,
        send_cw,
        recv_cw,
        send_ccw,
        recv_ccw,
    ):
        my_x = lax.axis_index("x")
        my_y = lax.axis_index("y")
        my_z = lax.axis_index("z")
        left = (my_z - 1) % N_DEV
        right = (my_z + 1) % N_DEV

        barrier_sem = pltpu.get_barrier_semaphore()
        for nbr in (left, right):
            pl.semaphore_signal(
                barrier_sem,
                inc=1,
                device_id=(my_x, my_y, nbr),
                device_id_type=pl.DeviceIdType.MESH,
            )
        pl.semaphore_wait(barrier_sem, 2)

        comm_cw[0, :, :] = x_ref[pl.ds(my_z * m_c, m_c), 0:n_h]
        comm_ccw[0, :, :] = x_ref[pl.ds(my_z * m_c, m_c), n_h:n]

        def hop(h, src_cw_slot, dst_cw_slot):
            cw = pltpu.make_async_remote_copy(
                src_ref=comm_cw.at[h],
                dst_ref=comm_cw.at[h + 1],
                send_sem=send_cw.at[h],
                recv_sem=recv_cw.at[h],
                device_id=(my_x, my_y, right),
                device_id_type=pl.DeviceIdType.MESH,
            )
            ccw = pltpu.make_async_remote_copy(
                src_ref=comm_ccw.at[h],
                dst_ref=comm_ccw.at[h + 1],
                send_sem=send_ccw.at[h],
                recv_sem=recv_ccw.at[h],
                device_id=(my_x, my_y, left),
                device_id_type=pl.DeviceIdType.MESH,
            )
            cw.start()
            ccw.start()
            return cw, ccw

        for h in range(N_DEV - 1):
            cw, ccw = hop(h, h, h + 1)
            cw.wait()
            c = (my_z - 1 - h) % N_DEV
            comm_cw[h + 1, :, :] = (
                comm_cw[h + 1, :, :] + x_ref[pl.ds(c * m_c, m_c), 0:n_h]
            )
            ccw.wait()
            c = (my_z + 1 + h) % N_DEV
            comm_ccw[h + 1, :, :] = (
                comm_ccw[h + 1, :, :] + x_ref[pl.ds(c * m_c, m_c), n_h:n]
            )

        out_ref[pl.ds(right * m_c, m_c), 0:n_h] = comm_cw[N_DEV - 1, :, :]
        out_ref[pl.ds(left * m_c, m_c), n_h:n] = comm_ccw[N_DEV - 1, :, :]

        for h in range(N_DEV - 1):
            g = (N_DEV - 1) + h
            cw, ccw = hop(g, g, g + 1)
            cw.wait()
            c = (my_z - h) % N_DEV
            out_ref[pl.ds(c * m_c, m_c), 0:n_h] = comm_cw[g + 1, :, :]
            ccw.wait()
            c = (my_z + h) % N_DEV
            out_ref[pl.ds(c * m_c, m_c), n_h:n] = comm_ccw[g + 1, :, :]

    return pl.pallas_call(
        body,
        out_shape=jax.ShapeDtypeStruct((m, n), jnp.float32),
        in_specs=[pl.BlockSpec(memory_space=pltpu.VMEM)],
        out_specs=pl.BlockSpec(memory_space=pltpu.VMEM),
        scratch_shapes=[
            pltpu.VMEM((N_HOPS + 1, m_c, n_h), jnp.float32),
            pltpu.VMEM((N_HOPS + 1, m_c, n_h), jnp.float32),
            pltpu.SemaphoreType.DMA((N_HOPS,)),
            pltpu.SemaphoreType.DMA((N_HOPS,)),
            pltpu.SemaphoreType.DMA((N_HOPS,)),
            pltpu.SemaphoreType.DMA((N_HOPS,)),
        ],
        compiler_params=pltpu.CompilerParams(collective_id=0),
    )(x)


# device time: 72677 ns/iter; 1.1737x vs baseline; 1.1737x over previous
import jax
import jax.numpy as jnp
from jax import lax
from jax.experimental import pallas as pl
from jax.experimental.pallas import tpu as pltpu

N_Z = 4
N_LANE = 4
S = 2
N_XY_HOPS = N_LANE - 1


def kernel(x):
    m, n = x.shape
    m_q = m // N_LANE
    n_s = n // S
    n_h = n // 2

    def body(
        x_ref,
        out_ref,
        pref_in,
        pref_out,
        suff_in,
        suff_out,
        cw_buf,
        ccw_buf,
        pref_send,
        pref_recv,
        suff_send,
        suff_recv,
        cw_send,
        cw_recv,
        ccw_send,
        ccw_recv,
    ):
        my_x = lax.axis_index("x")
        my_y = lax.axis_index("y")
        my_z = lax.axis_index("z")

        r = 3 * my_x + my_y - 2 * my_x * my_y

        def ring_coords(k):
            kx = jnp.where(k >= 2, 1, 0)
            ky = jnp.where((k == 1) | (k == 2), 1, 0)
            return kx, ky

        nxt_x, nxt_y = ring_coords((r + 1) % N_LANE)
        prv_x, prv_y = ring_coords((r - 1) % N_LANE)

        barrier_sem = pltpu.get_barrier_semaphore()

        def bsig(dev):
            pl.semaphore_signal(
                barrier_sem, inc=1, device_id=dev,
                device_id_type=pl.DeviceIdType.MESH,
            )

        bsig((nxt_x, nxt_y, my_z))
        bsig((prv_x, prv_y, my_z))

        @pl.when(my_z < N_Z - 1)
        def _():
            bsig((my_x, my_y, my_z + 1))

        @pl.when(my_z > 0)
        def _():
            bsig((my_x, my_y, my_z - 1))

        pl.semaphore_wait(barrier_sem, 2)

        @pl.when(my_z < N_Z - 1)
        def _():
            pl.semaphore_wait(barrier_sem, 1)

        @pl.when(my_z > 0)
        def _():
            pl.semaphore_wait(barrier_sem, 1)

        q_rows = pl.ds(r * m_q, m_q)

        def pref_rdma(s):
            cs = pl.ds(s * n_s, n_s)
            return pltpu.make_async_remote_copy(
                src_ref=pref_out.at[:, cs],
                dst_ref=pref_in.at[:, cs],
                send_sem=pref_send.at[s],
                recv_sem=pref_recv.at[s],
                device_id=(my_x, my_y, my_z + 1),
                device_id_type=pl.DeviceIdType.MESH,
            )

        def suff_rdma(s):
            cs = pl.ds(s * n_s, n_s)
            return pltpu.make_async_remote_copy(
                src_ref=suff_out.at[:, cs],
                dst_ref=suff_in.at[:, cs],
                send_sem=suff_send.at[s],
                recv_sem=suff_recv.at[s],
                device_id=(my_x, my_y, my_z - 1),
                device_id_type=pl.DeviceIdType.MESH,
            )

        def prefix_block(s):
            cs = pl.ds(s * n_s, n_s)

            @pl.when(my_z == 0)
            def _():
                pref_out[:, cs] = x_ref[q_rows, cs]

            @pl.when(my_z > 0)
            def _():
                pref_rdma(s).wait_recv()
                pref_out[:, cs] = pref_in[:, cs] + x_ref[q_rows, cs]

            @pl.when(my_z < N_Z - 1)
            def _():
                pref_rdma(s).start()

        def suffix_block(s):
            cs = pl.ds(s * n_s, n_s)

            @pl.when(my_z == N_Z - 1)
            def _():
                suff_out[:, cs] = x_ref[q_rows, cs]

            @pl.when(my_z < N_Z - 1)
            def _():
                suff_rdma(s).wait_recv()
                suff_out[:, cs] = suff_in[:, cs] + x_ref[q_rows, cs]

            @pl.when(my_z > 0)
            def _():
                suff_rdma(s).start()

        @pl.when(my_z <= 1)
        def _():
            for s in range(S):
                prefix_block(s)
                suffix_block(s)

        @pl.when(my_z >= 2)
        def _():
            for s in range(S):
                suffix_block(s)
                prefix_block(s)

        @pl.when(my_z < N_Z - 1)
        def _():
            out_ref[q_rows, :] = pref_out[:, :] + suff_in[:, :]

        @pl.when(my_z == N_Z - 1)
        def _():
            out_ref[q_rows, :] = pref_out[:, :]

        cw_buf[0, :, :] = out_ref[q_rows, 0:n_h]
        ccw_buf[0, :, :] = out_ref[q_rows, n_h:n]

        for h in range(N_XY_HOPS):
            cw = pltpu.make_async_remote_copy(
                src_ref=cw_buf.at[h],
                dst_ref=cw_buf.at[h + 1],
                send_sem=cw_send.at[h],
                recv_sem=cw_recv.at[h],
                device_id=(nxt_x, nxt_y, my_z),
                device_id_type=pl.DeviceIdType.MESH,
            )
            ccw = pltpu.make_async_remote_copy(
                src_ref=ccw_buf.at[h],
                dst_ref=ccw_buf.at[h + 1],
                send_sem=ccw_send.at[h],
                recv_sem=ccw_recv.at[h],
                device_id=(prv_x, prv_y, my_z),
                device_id_type=pl.DeviceIdType.MESH,
            )
            cw.start()
            ccw.start()
            cw.wait()
            q = (r - 1 - h) % N_LANE
            out_ref[pl.ds(q * m_q, m_q), 0:n_h] = cw_buf[h + 1, :, :]
            ccw.wait()
            q = (r + 1 + h) % N_LANE
            out_ref[pl.ds(q * m_q, m_q), n_h:n] = ccw_buf[h + 1, :, :]

        @pl.when(my_z < N_Z - 1)
        def _():
            for s in range(S):
                pref_rdma(s).wait_send()

        @pl.when(my_z > 0)
        def _():
            for s in range(S):
                suff_rdma(s).wait_send()

    return pl.pallas_call(
        body,
        out_shape=jax.ShapeDtypeStruct((m, n), jnp.float32),
        in_specs=[pl.BlockSpec(memory_space=pltpu.VMEM)],
        out_specs=pl.BlockSpec(memory_space=pltpu.VMEM),
        scratch_shapes=[
            pltpu.VMEM((m_q, n), jnp.float32),
            pltpu.VMEM((m_q, n), jnp.float32),
            pltpu.VMEM((m_q, n), jnp.float32),
            pltpu.VMEM((m_q, n), jnp.float32),
            pltpu.VMEM((N_XY_HOPS + 1, m_q, n_h), jnp.float32),
            pltpu.VMEM((N_XY_HOPS + 1, m_q, n_h), jnp.float32),
            pltpu.SemaphoreType.DMA((S,)),
            pltpu.SemaphoreType.DMA((S,)),
            pltpu.SemaphoreType.DMA((S,)),
            pltpu.SemaphoreType.DMA((S,)),
            pltpu.SemaphoreType.DMA((N_XY_HOPS,)),
            pltpu.SemaphoreType.DMA((N_XY_HOPS,)),
            pltpu.SemaphoreType.DMA((N_XY_HOPS,)),
            pltpu.SemaphoreType.DMA((N_XY_HOPS,)),
        ],
        compiler_params=pltpu.CompilerParams(collective_id=0),
    )(x)
